# baseline (speedup 1.0000x reference)
import functools

import jax
import jax.numpy as jnp
from jax import lax
from jax.experimental import pallas as pl
from jax.experimental.pallas import tpu as pltpu
from jax.experimental.pallas import tpu_sc as plsc

N = 10000
E = 320000
D_IN = 128
D_HID = 64
D_OUT = 128

NC = 2
NS = 16
EB = 125
E_ROWS = E // EB
NB = E_ROWS // NC // NS


@functools.lru_cache(maxsize=None)
def _mesh():
  return plsc.VectorSubcoreMesh(
      core_axis_name="c", subcore_axis_name="s", num_cores=NC, num_subcores=NS)


def _zero_slices(zb_hbm, dst_sh, s, d):
  for k in range(5):
    pltpu.sync_copy(zb_hbm.at[:, pl.ds(0, d)] if d < 128 else zb_hbm,
                    dst_sh.at[pl.ds(s * 625 + k * EB, EB)])


def _deg_body(dst_hbm, zb_hbm, ones_hbm, out_hbm, deg_sh, ones_v, idx_v):
  c = lax.axis_index("c")
  s = lax.axis_index("s")
  base = c * (E_ROWS // NC) + s * NB
  pltpu.sync_copy(dst_hbm.at[pl.ds(base, NB)], idx_v)
  _zero_slices(zb_hbm, deg_sh, s, 16)
  pltpu.sync_copy(ones_hbm, ones_v)
  plsc.subcore_barrier()

  def blk(j, _):
    pltpu.sync_copy(ones_v, deg_sh.at[idx_v.at[j]], add=True)
    return 0

  lax.fori_loop(0, NB, blk, 0)
  plsc.subcore_barrier()

  @pl.when(s == 0)
  def _():
    pltpu.sync_copy(deg_sh, out_hbm.at[:, pl.ds(16 * c, 16)])


@functools.lru_cache(maxsize=None)
def _deg_kernel():
  return pl.kernel(
      _deg_body,
      out_type=jax.ShapeDtypeStruct((N, 128), jnp.float32),
      mesh=_mesh(),
      compiler_params=pltpu.CompilerParams(use_tc_tiling_on_sc=False),
      scratch_types=[
          pltpu.VMEM_SHARED((N, 16), jnp.float32),
          pltpu.VMEM((EB, 16), jnp.float32),
          pltpu.VMEM((NB, EB), jnp.int32),
      ],
  )


def _pair_pipeline(gather_ref, agg_sh, src_v, dst_v, rows0, rows1,
                   sem0, sem1, j_lo, j_hi):
  pltpu.async_copy(gather_ref.at[src_v.at[j_lo]], rows0, sem0)

  def pair(k, _):
    j0 = j_lo + 2 * k
    j1 = j0 + 1
    pltpu.async_copy(gather_ref.at[src_v.at[j1]], rows1, sem1)
    pltpu.make_async_copy(gather_ref.at[src_v.at[j0]], rows0, sem0).wait()
    pltpu.sync_copy(rows0, agg_sh.at[dst_v.at[j0]], add=True)

    @pl.when(j0 + 2 < j_hi)
    def _():
      pltpu.async_copy(gather_ref.at[src_v.at[j0 + 2]], rows0, sem0)

    pltpu.make_async_copy(gather_ref.at[src_v.at[j1]], rows1, sem1).wait()
    pltpu.sync_copy(rows1, agg_sh.at[dst_v.at[j1]], add=True)
    return 0

  lax.fori_loop(0, (j_hi - j_lo) // 2, pair, 0)


def _agg_body(d, crows, spmem_blocks, p_hbm, zb_hbm, src_hbm, dst_hbm,
              out_hbm, agg_sh, src_v, dst_v, rows0, rows1, sem0, sem1,
              *maybe_p_sh):
  c = lax.axis_index("c")
  s = lax.axis_index("s")
  base = c * (E_ROWS // NC) + s * NB

  for t in range(NB // crows):
    pltpu.sync_copy(src_hbm.at[pl.ds(base + t * crows, crows)], src_v)
    pltpu.sync_copy(dst_hbm.at[pl.ds(base + t * crows, crows)], dst_v)
    if t == 0:
      _zero_slices(zb_hbm, agg_sh, s, d)
      if spmem_blocks:
        @pl.when(s == 0)
        def _():
          pltpu.sync_copy(p_hbm, maybe_p_sh[0])
      plsc.subcore_barrier()

    if spmem_blocks:
      _pair_pipeline(maybe_p_sh[0], agg_sh, src_v, dst_v, rows0, rows1,
                     sem0, sem1, 0, spmem_blocks)
    _pair_pipeline(p_hbm, agg_sh, src_v, dst_v, rows0, rows1,
                   sem0, sem1, spmem_blocks, crows)
  plsc.subcore_barrier()

  @pl.when(s == 0)
  def _():
    if d == 128:
      pltpu.sync_copy(agg_sh, out_hbm.at[pl.ds(c * N, N)])
    else:
      pltpu.sync_copy(agg_sh, out_hbm.at[:, pl.ds(d * c, d)])


@functools.lru_cache(maxsize=None)
def _make_agg_kernel(d, crows, spmem_blocks):
  scratch = [
      pltpu.VMEM_SHARED((N, d), jnp.float32),
      pltpu.VMEM((crows, EB), jnp.int32),
      pltpu.VMEM((crows, EB), jnp.int32),
      pltpu.VMEM((EB, d), jnp.float32),
      pltpu.VMEM((EB, d), jnp.float32),
      pltpu.SemaphoreType.DMA,
      pltpu.SemaphoreType.DMA,
  ]
  if spmem_blocks:
    scratch.append(pltpu.VMEM_SHARED((N, d), jnp.float32))
  out_shape = (NC * N, d) if d == 128 else (N, NC * d)
  return pl.kernel(
      functools.partial(_agg_body, d, crows, spmem_blocks),
      out_type=jax.ShapeDtypeStruct(out_shape, jnp.float32),
      mesh=_mesh(),
      compiler_params=pltpu.CompilerParams(use_tc_tiling_on_sc=False),
      scratch_types=scratch,
  )


_BN = 1000
_GRID = N // _BN


def _k1a_body(x_ref, w_ref, h_ref):
  h_ref[...] = jnp.dot(x_ref[...], w_ref[...],
                       preferred_element_type=jnp.float32)


def _tc1a(x, W1):
  return pl.pallas_call(
      _k1a_body,
      grid=(_GRID,),
      in_specs=[
          pl.BlockSpec((_BN, D_IN), lambda i: (i, 0)),
          pl.BlockSpec((D_IN, D_HID), lambda i: (0, 0)),
      ],
      out_specs=pl.BlockSpec((_BN, D_HID), lambda i: (i, 0)),
      out_shape=jax.ShapeDtypeStruct((N, D_HID), jnp.float32),
  )(x, W1)


def _k1b_body(degp_ref, h_ref, p_ref, dis_ref):
  deg = degp_ref[:, :1] + degp_ref[:, 16:17] + 1.0
  dis = lax.rsqrt(deg)
  dis_ref[...] = dis
  p_ref[...] = h_ref[...] * dis


def _tc1b(degp, h1):
  return pl.pallas_call(
      _k1b_body,
      grid=(_GRID,),
      in_specs=[
          pl.BlockSpec((_BN, 128), lambda i: (i, 0)),
          pl.BlockSpec((_BN, D_HID), lambda i: (i, 0)),
      ],
      out_specs=[
          pl.BlockSpec((_BN, D_HID), lambda i: (i, 0)),
          pl.BlockSpec((_BN, 1), lambda i: (i, 0)),
      ],
      out_shape=[
          jax.ShapeDtypeStruct((N, D_HID), jnp.float32),
          jax.ShapeDtypeStruct((N, 1), jnp.float32),
      ],
  )(degp, h1)


def _k2_body(agg_ref, p_ref, dis_ref, b_ref, w_ref, out_ref):
  dis = dis_ref[...]
  agg = agg_ref[:, :D_HID] + agg_ref[:, D_HID:]
  z = dis * (agg + p_ref[...]) + b_ref[...]
  z = jnp.maximum(z, 0.0)
  h = jnp.dot(z, w_ref[...], preferred_element_type=jnp.float32)
  out_ref[...] = h * dis


def _tc2(agg, p, dis, b, W, d_out):
  return pl.pallas_call(
      _k2_body,
      grid=(_GRID,),
      in_specs=[
          pl.BlockSpec((_BN, NC * D_HID), lambda i: (i, 0)),
          pl.BlockSpec((_BN, D_HID), lambda i: (i, 0)),
          pl.BlockSpec((_BN, 1), lambda i: (i, 0)),
          pl.BlockSpec((1, D_HID), lambda i: (0, 0)),
          pl.BlockSpec((D_HID, d_out), lambda i: (0, 0)),
      ],
      out_specs=pl.BlockSpec((_BN, d_out), lambda i: (i, 0)),
      out_shape=jax.ShapeDtypeStruct((N, d_out), jnp.float32),
  )(agg, p, dis, b, W)


def _k4_body(agga_ref, aggb_ref, p_ref, dis_ref, b_ref, out_ref):
  out_ref[...] = dis_ref[...] * (
      agga_ref[...] + aggb_ref[...] + p_ref[...]) + b_ref[...]


def _tc4(agg3, p3, dis, b3):
  return pl.pallas_call(
      _k4_body,
      grid=(_GRID,),
      in_specs=[
          pl.BlockSpec((_BN, D_OUT), lambda i: (i, 0)),
          pl.BlockSpec((_BN, D_OUT), lambda i: (i + _GRID, 0)),
          pl.BlockSpec((_BN, D_OUT), lambda i: (i, 0)),
          pl.BlockSpec((_BN, 1), lambda i: (i, 0)),
          pl.BlockSpec((1, D_OUT), lambda i: (0, 0)),
      ],
      out_specs=pl.BlockSpec((_BN, D_OUT), lambda i: (i, 0)),
      out_shape=jax.ShapeDtypeStruct((N, D_OUT), jnp.float32),
  )(agg3, agg3, p3, dis, b3)


def kernel(x, edge_index, W1, b1, W2, b2, W3, b3):
  ei = edge_index.astype(jnp.int32)
  src = ei[0].reshape(E_ROWS, EB)
  dst = ei[1].reshape(E_ROWS, EB)

  zb = jnp.zeros((EB, 128), jnp.float32)
  ones125 = jnp.ones((EB, 16), jnp.float32)

  degp = _deg_kernel()(dst, zb, ones125)

  h1 = _tc1a(x, W1)
  p1, dis = _tc1b(degp, h1)
  agg1 = _make_agg_kernel(D_HID, 80, 0)(p1, zb, src, dst)
  p2 = _tc2(agg1, p1, dis, b1.reshape(1, D_HID), W2, D_HID)
  agg2 = _make_agg_kernel(D_HID, 80, 0)(p2, zb, src, dst)
  p3 = _tc2(agg2, p2, dis, b2.reshape(1, D_HID), W3, D_OUT)
  agg3 = _make_agg_kernel(D_OUT, 40, 0)(p3, zb, src, dst)
  return _tc4(agg3, p3, dis, b3.reshape(1, D_OUT))


# speedup vs baseline: 1.0416x; 1.0416x over previous
import functools

import jax
import jax.numpy as jnp
from jax import lax
from jax.experimental import pallas as pl
from jax.experimental.pallas import tpu as pltpu
from jax.experimental.pallas import tpu_sc as plsc

N = 10000
E = 320000
D_IN = 128
D_HID = 64
D_OUT = 128

NC = 2
NS = 16
EB = 125
E_ROWS = E // EB
NB = E_ROWS // NC // NS


@functools.lru_cache(maxsize=None)
def _mesh():
  return plsc.VectorSubcoreMesh(
      core_axis_name="c", subcore_axis_name="s", num_cores=NC, num_subcores=NS)


def _deg_body(dst_hbm, zeros_hbm, ones_hbm, out_hbm, deg_sh, ones_v, idx_v):
  c = lax.axis_index("c")
  s = lax.axis_index("s")
  base = c * (E_ROWS // NC) + s * NB
  pltpu.sync_copy(dst_hbm.at[pl.ds(base, NB)], idx_v)
  @pl.when(s == 0)
  def _():
    pltpu.sync_copy(zeros_hbm, deg_sh)
  pltpu.sync_copy(ones_hbm, ones_v)
  plsc.subcore_barrier()

  def blk(j, _):
    pltpu.sync_copy(ones_v, deg_sh.at[idx_v.at[j]], add=True)
    return 0

  lax.fori_loop(0, NB, blk, 0)
  plsc.subcore_barrier()

  @pl.when(s == 0)
  def _():
    pltpu.sync_copy(deg_sh, out_hbm.at[:, pl.ds(16 * c, 16)])


@functools.lru_cache(maxsize=None)
def _deg_kernel():
  return pl.kernel(
      _deg_body,
      out_type=jax.ShapeDtypeStruct((N, 128), jnp.float32),
      mesh=_mesh(),
      compiler_params=pltpu.CompilerParams(use_tc_tiling_on_sc=False),
      scratch_types=[
          pltpu.VMEM_SHARED((N, 16), jnp.float32),
          pltpu.VMEM((EB, 16), jnp.float32),
          pltpu.VMEM((NB, EB), jnp.int32),
      ],
  )


def _pair_pipeline(gather_ref, agg_sh, src_v, dst_v, rows0, rows1,
                   sem0, sem1, j_lo, j_hi):
  pltpu.async_copy(gather_ref.at[src_v.at[j_lo]], rows0, sem0)

  def pair(k, _):
    j0 = j_lo + 2 * k
    j1 = j0 + 1
    pltpu.async_copy(gather_ref.at[src_v.at[j1]], rows1, sem1)
    pltpu.make_async_copy(gather_ref.at[src_v.at[j0]], rows0, sem0).wait()
    pltpu.sync_copy(rows0, agg_sh.at[dst_v.at[j0]], add=True)

    @pl.when(j0 + 2 < j_hi)
    def _():
      pltpu.async_copy(gather_ref.at[src_v.at[j0 + 2]], rows0, sem0)

    pltpu.make_async_copy(gather_ref.at[src_v.at[j1]], rows1, sem1).wait()
    pltpu.sync_copy(rows1, agg_sh.at[dst_v.at[j1]], add=True)
    return 0

  lax.fori_loop(0, (j_hi - j_lo) // 2, pair, 0)


def _agg_body(d, crows, spmem_blocks, p_hbm, zeros_hbm, src_hbm, dst_hbm,
              out_hbm, agg_sh, src_v, dst_v, rows0, rows1, sem0, sem1,
              *maybe_p_sh):
  c = lax.axis_index("c")
  s = lax.axis_index("s")
  base = c * (E_ROWS // NC) + s * NB

  for t in range(NB // crows):
    pltpu.sync_copy(src_hbm.at[pl.ds(base + t * crows, crows)], src_v)
    pltpu.sync_copy(dst_hbm.at[pl.ds(base + t * crows, crows)], dst_v)
    if t == 0:
      @pl.when(s == 0)
      def _():
        pltpu.sync_copy(zeros_hbm, agg_sh)
        if spmem_blocks:
          pltpu.sync_copy(p_hbm, maybe_p_sh[0])
      plsc.subcore_barrier()

    if spmem_blocks:
      _pair_pipeline(maybe_p_sh[0], agg_sh, src_v, dst_v, rows0, rows1,
                     sem0, sem1, 0, spmem_blocks)
    _pair_pipeline(p_hbm, agg_sh, src_v, dst_v, rows0, rows1,
                   sem0, sem1, spmem_blocks, crows)
  plsc.subcore_barrier()

  @pl.when(s == 0)
  def _():
    if d == 128:
      pltpu.sync_copy(agg_sh, out_hbm.at[pl.ds(c * N, N)])
    else:
      pltpu.sync_copy(agg_sh, out_hbm.at[:, pl.ds(d * c, d)])


@functools.lru_cache(maxsize=None)
def _make_agg_kernel(d, crows, spmem_blocks):
  scratch = [
      pltpu.VMEM_SHARED((N, d), jnp.float32),
      pltpu.VMEM((crows, EB), jnp.int32),
      pltpu.VMEM((crows, EB), jnp.int32),
      pltpu.VMEM((EB, d), jnp.float32),
      pltpu.VMEM((EB, d), jnp.float32),
      pltpu.SemaphoreType.DMA,
      pltpu.SemaphoreType.DMA,
  ]
  if spmem_blocks:
    scratch.append(pltpu.VMEM_SHARED((N, d), jnp.float32))
  out_shape = (NC * N, d) if d == 128 else (N, NC * d)
  return pl.kernel(
      functools.partial(_agg_body, d, crows, spmem_blocks),
      out_type=jax.ShapeDtypeStruct(out_shape, jnp.float32),
      mesh=_mesh(),
      compiler_params=pltpu.CompilerParams(use_tc_tiling_on_sc=False),
      scratch_types=scratch,
  )


_BN = 1000
_GRID = N // _BN


def _k1a_body(x_ref, w_ref, h_ref):
  h_ref[...] = jnp.dot(x_ref[...], w_ref[...],
                       preferred_element_type=jnp.float32)


def _tc1a(x, W1):
  return pl.pallas_call(
      _k1a_body,
      grid=(_GRID,),
      in_specs=[
          pl.BlockSpec((_BN, D_IN), lambda i: (i, 0)),
          pl.BlockSpec((D_IN, D_HID), lambda i: (0, 0)),
      ],
      out_specs=pl.BlockSpec((_BN, D_HID), lambda i: (i, 0)),
      out_shape=jax.ShapeDtypeStruct((N, D_HID), jnp.float32),
  )(x, W1)


def _k1b_body(degp_ref, h_ref, p_ref, dis_ref):
  deg = degp_ref[:, :1] + degp_ref[:, 16:17] + 1.0
  dis = lax.rsqrt(deg)
  dis_ref[...] = dis
  p_ref[...] = h_ref[...] * dis


def _tc1b(degp, h1):
  return pl.pallas_call(
      _k1b_body,
      grid=(_GRID,),
      in_specs=[
          pl.BlockSpec((_BN, 128), lambda i: (i, 0)),
          pl.BlockSpec((_BN, D_HID), lambda i: (i, 0)),
      ],
      out_specs=[
          pl.BlockSpec((_BN, D_HID), lambda i: (i, 0)),
          pl.BlockSpec((_BN, 1), lambda i: (i, 0)),
      ],
      out_shape=[
          jax.ShapeDtypeStruct((N, D_HID), jnp.float32),
          jax.ShapeDtypeStruct((N, 1), jnp.float32),
      ],
  )(degp, h1)


def _k2_body(agg_ref, p_ref, dis_ref, b_ref, w_ref, out_ref):
  dis = dis_ref[...]
  agg = agg_ref[:, :D_HID] + agg_ref[:, D_HID:]
  z = dis * (agg + p_ref[...]) + b_ref[...]
  z = jnp.maximum(z, 0.0)
  h = jnp.dot(z, w_ref[...], preferred_element_type=jnp.float32)
  out_ref[...] = h * dis


def _tc2(agg, p, dis, b, W, d_out):
  return pl.pallas_call(
      _k2_body,
      grid=(_GRID,),
      in_specs=[
          pl.BlockSpec((_BN, NC * D_HID), lambda i: (i, 0)),
          pl.BlockSpec((_BN, D_HID), lambda i: (i, 0)),
          pl.BlockSpec((_BN, 1), lambda i: (i, 0)),
          pl.BlockSpec((1, D_HID), lambda i: (0, 0)),
          pl.BlockSpec((D_HID, d_out), lambda i: (0, 0)),
      ],
      out_specs=pl.BlockSpec((_BN, d_out), lambda i: (i, 0)),
      out_shape=jax.ShapeDtypeStruct((N, d_out), jnp.float32),
  )(agg, p, dis, b, W)


def _k4_body(agga_ref, aggb_ref, p_ref, dis_ref, b_ref, out_ref):
  out_ref[...] = dis_ref[...] * (
      agga_ref[...] + aggb_ref[...] + p_ref[...]) + b_ref[...]


def _tc4(agg3, p3, dis, b3):
  return pl.pallas_call(
      _k4_body,
      grid=(_GRID,),
      in_specs=[
          pl.BlockSpec((_BN, D_OUT), lambda i: (i, 0)),
          pl.BlockSpec((_BN, D_OUT), lambda i: (i + _GRID, 0)),
          pl.BlockSpec((_BN, D_OUT), lambda i: (i, 0)),
          pl.BlockSpec((_BN, 1), lambda i: (i, 0)),
          pl.BlockSpec((1, D_OUT), lambda i: (0, 0)),
      ],
      out_specs=pl.BlockSpec((_BN, D_OUT), lambda i: (i, 0)),
      out_shape=jax.ShapeDtypeStruct((N, D_OUT), jnp.float32),
  )(agg3, agg3, p3, dis, b3)


def kernel(x, edge_index, W1, b1, W2, b2, W3, b3):
  ei = edge_index.astype(jnp.int32)
  src = ei[0].reshape(E_ROWS, EB)
  dst = ei[1].reshape(E_ROWS, EB)

  zeros64 = jnp.zeros((N, D_HID), jnp.float32)
  zeros128 = jnp.zeros((N, D_OUT), jnp.float32)
  zeros16 = jnp.zeros((N, 16), jnp.float32)
  ones125 = jnp.ones((EB, 16), jnp.float32)

  degp = _deg_kernel()(dst, zeros16, ones125)

  h1 = _tc1a(x, W1)
  p1, dis = _tc1b(degp, h1)
  agg1 = _make_agg_kernel(D_HID, 80, 0)(p1, zeros64, src, dst)
  p2 = _tc2(agg1, p1, dis, b1.reshape(1, D_HID), W2, D_HID)
  agg2 = _make_agg_kernel(D_HID, 80, 0)(p2, zeros64, src, dst)
  p3 = _tc2(agg2, p2, dis, b2.reshape(1, D_HID), W3, D_OUT)
  agg3 = _make_agg_kernel(D_OUT, 40, 0)(p3, zeros128, src, dst)
  return _tc4(agg3, p3, dis, b3.reshape(1, D_OUT))


# speedup vs baseline: 1.0835x; 1.0403x over previous
import functools

import jax
import jax.numpy as jnp
from jax import lax
from jax.experimental import pallas as pl
from jax.experimental.pallas import tpu as pltpu
from jax.experimental.pallas import tpu_sc as plsc

N = 10000
E = 320000
D_IN = 128
D_HID = 64
D_OUT = 128

NC = 2
NS = 16
EB = 125
E_ROWS = E // EB
NB = E_ROWS // NC // NS


@functools.lru_cache(maxsize=None)
def _mesh():
  return plsc.VectorSubcoreMesh(
      core_axis_name="c", subcore_axis_name="s", num_cores=NC, num_subcores=NS)


def _deg_body(dst_hbm, zeros_hbm, ones_hbm, out_hbm, deg_sh, ones_v, idx_v):
  c = lax.axis_index("c")
  s = lax.axis_index("s")
  base = c * (E_ROWS // NC) + s * NB
  pltpu.sync_copy(dst_hbm.at[pl.ds(base, NB)], idx_v)
  @pl.when(s == 0)
  def _():
    pltpu.sync_copy(zeros_hbm, deg_sh)
  pltpu.sync_copy(ones_hbm, ones_v)
  plsc.subcore_barrier()

  def blk(j, _):
    pltpu.sync_copy(ones_v, deg_sh.at[idx_v.at[j]], add=True)
    return 0

  lax.fori_loop(0, NB, blk, 0)
  plsc.subcore_barrier()

  @pl.when(s == 0)
  def _():
    pltpu.sync_copy(deg_sh, out_hbm.at[:, pl.ds(16 * c, 16)])


@functools.lru_cache(maxsize=None)
def _deg_kernel():
  return pl.kernel(
      _deg_body,
      out_type=jax.ShapeDtypeStruct((N, 128), jnp.float32),
      mesh=_mesh(),
      compiler_params=pltpu.CompilerParams(use_tc_tiling_on_sc=False),
      scratch_types=[
          pltpu.VMEM_SHARED((N, 16), jnp.float32),
          pltpu.VMEM((EB, 16), jnp.float32),
          pltpu.VMEM((NB, EB), jnp.int32),
      ],
  )


def _pair_pipeline(gather_ref, agg_sh, src_v, dst_v, rows0, rows1,
                   sem0, sem1, j_lo, j_hi):
  pltpu.async_copy(gather_ref.at[src_v.at[j_lo]], rows0, sem0)

  def pair(k, _):
    j0 = j_lo + 2 * k
    j1 = j0 + 1
    pltpu.async_copy(gather_ref.at[src_v.at[j1]], rows1, sem1)
    pltpu.make_async_copy(gather_ref.at[src_v.at[j0]], rows0, sem0).wait()
    pltpu.sync_copy(rows0, agg_sh.at[dst_v.at[j0]], add=True)

    @pl.when(j0 + 2 < j_hi)
    def _():
      pltpu.async_copy(gather_ref.at[src_v.at[j0 + 2]], rows0, sem0)

    pltpu.make_async_copy(gather_ref.at[src_v.at[j1]], rows1, sem1).wait()
    pltpu.sync_copy(rows1, agg_sh.at[dst_v.at[j1]], add=True)
    return 0

  lax.fori_loop(0, (j_hi - j_lo) // 2, pair, 0)


def _agg_body(d, crows, spmem_blocks, p_hbm, zeros_hbm, src_hbm, dst_hbm,
              out_hbm, agg_sh, src_v, dst_v, rows0, rows1, sem0, sem1,
              *maybe_p_sh):
  c = lax.axis_index("c")
  s = lax.axis_index("s")
  base = c * (E_ROWS // NC) + s * NB

  for t in range(NB // crows):
    pltpu.sync_copy(src_hbm.at[pl.ds(base + t * crows, crows)], src_v)
    pltpu.sync_copy(dst_hbm.at[pl.ds(base + t * crows, crows)], dst_v)
    if t == 0:
      @pl.when(s == 0)
      def _():
        pltpu.sync_copy(zeros_hbm, agg_sh)
        if spmem_blocks:
          pltpu.sync_copy(p_hbm, maybe_p_sh[0])
      plsc.subcore_barrier()

    if spmem_blocks:
      _pair_pipeline(maybe_p_sh[0], agg_sh, src_v, dst_v, rows0, rows1,
                     sem0, sem1, 0, spmem_blocks)
    if d == 128:
      _pair_pipeline(p_hbm, agg_sh, src_v, dst_v, rows0, rows1,
                     sem0, sem1, spmem_blocks, crows)
    else:
      bufs = (rows0, rows1, maybe_p_sh[-2], maybe_p_sh[-1])
      gsems = sem0
      ssems = sem1

      def quad(k, _):
        j = 4 * k
        for i in range(4):
          @pl.when(k > 0)
          def _():
            pltpu.make_async_copy(
                bufs[i], agg_sh.at[dst_v.at[j + i - 4]], ssems.at[i]).wait()
          pltpu.async_copy(p_hbm.at[src_v.at[j + i]], bufs[i], gsems.at[i])
        for i in range(4):
          pltpu.make_async_copy(
              p_hbm.at[src_v.at[j + i]], bufs[i], gsems.at[i]).wait()
          pltpu.async_copy(bufs[i], agg_sh.at[dst_v.at[j + i]], ssems.at[i],
                           add=True)
        return 0

      lax.fori_loop(0, crows // 4, quad, 0)
      for i in range(4):
        pltpu.make_async_copy(
            bufs[i], agg_sh.at[dst_v.at[crows + i - 4]], ssems.at[i]).wait()
  plsc.subcore_barrier()

  @pl.when(s == 0)
  def _():
    if d == 128:
      pltpu.sync_copy(agg_sh, out_hbm.at[pl.ds(c * N, N)])
    else:
      pltpu.sync_copy(agg_sh, out_hbm.at[:, pl.ds(d * c, d)])


@functools.lru_cache(maxsize=None)
def _make_agg_kernel(d, crows, spmem_blocks):
  if d == 128:
    sems = [pltpu.SemaphoreType.DMA, pltpu.SemaphoreType.DMA]
    extra = []
  else:
    sems = [pltpu.SemaphoreType.DMA((4,)), pltpu.SemaphoreType.DMA((4,))]
    extra = [pltpu.VMEM((EB, d), jnp.float32),
             pltpu.VMEM((EB, d), jnp.float32)]
  scratch = [
      pltpu.VMEM_SHARED((N, d), jnp.float32),
      pltpu.VMEM((crows, EB), jnp.int32),
      pltpu.VMEM((crows, EB), jnp.int32),
      pltpu.VMEM((EB, d), jnp.float32),
      pltpu.VMEM((EB, d), jnp.float32),
  ] + sems + extra
  if spmem_blocks:
    scratch.append(pltpu.VMEM_SHARED((N, d), jnp.float32))
  out_shape = (NC * N, d) if d == 128 else (N, NC * d)
  return pl.kernel(
      functools.partial(_agg_body, d, crows, spmem_blocks),
      out_type=jax.ShapeDtypeStruct(out_shape, jnp.float32),
      mesh=_mesh(),
      compiler_params=pltpu.CompilerParams(use_tc_tiling_on_sc=False),
      scratch_types=scratch,
  )


_BN = 1000
_GRID = N // _BN


def _k1a_body(x_ref, w_ref, h_ref):
  h_ref[...] = jnp.dot(x_ref[...], w_ref[...],
                       preferred_element_type=jnp.float32)


def _tc1a(x, W1):
  return pl.pallas_call(
      _k1a_body,
      grid=(_GRID,),
      in_specs=[
          pl.BlockSpec((_BN, D_IN), lambda i: (i, 0)),
          pl.BlockSpec((D_IN, D_HID), lambda i: (0, 0)),
      ],
      out_specs=pl.BlockSpec((_BN, D_HID), lambda i: (i, 0)),
      out_shape=jax.ShapeDtypeStruct((N, D_HID), jnp.float32),
  )(x, W1)


def _k1b_body(degp_ref, h_ref, p_ref, dis_ref):
  deg = degp_ref[:, :1] + degp_ref[:, 16:17] + 1.0
  dis = lax.rsqrt(deg)
  dis_ref[...] = dis
  p_ref[...] = h_ref[...] * dis


def _tc1b(degp, h1):
  return pl.pallas_call(
      _k1b_body,
      grid=(_GRID,),
      in_specs=[
          pl.BlockSpec((_BN, 128), lambda i: (i, 0)),
          pl.BlockSpec((_BN, D_HID), lambda i: (i, 0)),
      ],
      out_specs=[
          pl.BlockSpec((_BN, D_HID), lambda i: (i, 0)),
          pl.BlockSpec((_BN, 1), lambda i: (i, 0)),
      ],
      out_shape=[
          jax.ShapeDtypeStruct((N, D_HID), jnp.float32),
          jax.ShapeDtypeStruct((N, 1), jnp.float32),
      ],
  )(degp, h1)


def _k2_body(agg_ref, p_ref, dis_ref, b_ref, w_ref, out_ref):
  dis = dis_ref[...]
  agg = agg_ref[:, :D_HID] + agg_ref[:, D_HID:]
  z = dis * (agg + p_ref[...]) + b_ref[...]
  z = jnp.maximum(z, 0.0)
  h = jnp.dot(z, w_ref[...], preferred_element_type=jnp.float32)
  out_ref[...] = h * dis


def _tc2(agg, p, dis, b, W, d_out):
  return pl.pallas_call(
      _k2_body,
      grid=(_GRID,),
      in_specs=[
          pl.BlockSpec((_BN, NC * D_HID), lambda i: (i, 0)),
          pl.BlockSpec((_BN, D_HID), lambda i: (i, 0)),
          pl.BlockSpec((_BN, 1), lambda i: (i, 0)),
          pl.BlockSpec((1, D_HID), lambda i: (0, 0)),
          pl.BlockSpec((D_HID, d_out), lambda i: (0, 0)),
      ],
      out_specs=pl.BlockSpec((_BN, d_out), lambda i: (i, 0)),
      out_shape=jax.ShapeDtypeStruct((N, d_out), jnp.float32),
  )(agg, p, dis, b, W)


def _k4_body(agga_ref, aggb_ref, p_ref, dis_ref, b_ref, out_ref):
  out_ref[...] = dis_ref[...] * (
      agga_ref[...] + aggb_ref[...] + p_ref[...]) + b_ref[...]


def _tc4(agg3, p3, dis, b3):
  return pl.pallas_call(
      _k4_body,
      grid=(_GRID,),
      in_specs=[
          pl.BlockSpec((_BN, D_OUT), lambda i: (i, 0)),
          pl.BlockSpec((_BN, D_OUT), lambda i: (i + _GRID, 0)),
          pl.BlockSpec((_BN, D_OUT), lambda i: (i, 0)),
          pl.BlockSpec((_BN, 1), lambda i: (i, 0)),
          pl.BlockSpec((1, D_OUT), lambda i: (0, 0)),
      ],
      out_specs=pl.BlockSpec((_BN, D_OUT), lambda i: (i, 0)),
      out_shape=jax.ShapeDtypeStruct((N, D_OUT), jnp.float32),
  )(agg3, agg3, p3, dis, b3)


def kernel(x, edge_index, W1, b1, W2, b2, W3, b3):
  ei = edge_index.astype(jnp.int32)
  src = ei[0].reshape(E_ROWS, EB)
  dst = ei[1].reshape(E_ROWS, EB)

  zeros64 = jnp.zeros((N, D_HID), jnp.float32)
  zeros128 = jnp.zeros((N, D_OUT), jnp.float32)
  zeros16 = jnp.zeros((N, 16), jnp.float32)
  ones125 = jnp.ones((EB, 16), jnp.float32)

  degp = _deg_kernel()(dst, zeros16, ones125)

  h1 = _tc1a(x, W1)
  p1, dis = _tc1b(degp, h1)
  agg1 = _make_agg_kernel(D_HID, 80, 0)(p1, zeros64, src, dst)
  p2 = _tc2(agg1, p1, dis, b1.reshape(1, D_HID), W2, D_HID)
  agg2 = _make_agg_kernel(D_HID, 80, 0)(p2, zeros64, src, dst)
  p3 = _tc2(agg2, p2, dis, b2.reshape(1, D_HID), W3, D_OUT)
  agg3 = _make_agg_kernel(D_OUT, 40, 0)(p3, zeros128, src, dst)
  return _tc4(agg3, p3, dis, b3.reshape(1, D_OUT))
